# baseline (speedup 1.0000x reference)
import jax
import jax.numpy as jnp
from jax import lax
from jax.experimental import pallas as pl
from jax.experimental.pallas import tpu as pltpu
from jax.experimental.pallas import tpu_sc as plsc

_B = 128
_L = 2048
_C = 32
_CB = 128
_NCB = _L // _CB
_NW = 32
_PPW = _B * _B // _NW
_NCHUNK = 4
_CPAIRS = _PPW // _NCHUNK


def _prod_kernel(tcol_ref, trow_ref, lat_ref, y_ref, bins_ref):
    tom = jnp.abs(tcol_ref[...] - trow_ref[...]) + 1.0
    b = jnp.ceil(jnp.log2(tom) * 128.0)
    bins_ref[...] = jnp.clip(b, 0.0, float(_L)).astype(jnp.int32)

    k = jax.lax.broadcasted_iota(jnp.int32, (1, _L), 1).astype(jnp.float32)
    norm = jnp.exp2((k + 1.0) / 128.0) - jnp.exp2(k / 128.0)
    norm_last = jnp.exp2(jnp.float32(_L) / 128.0) - jnp.exp2(
        (jnp.float32(_L) - 1.0) / 128.0)
    y_ref[...] = lat_ref[...] * jnp.sqrt(norm_last / norm)


def _prod_call(latents, time_steps):
    t_f = time_steps.astype(jnp.float32)
    return pl.pallas_call(
        _prod_kernel,
        out_shape=(
            jax.ShapeDtypeStruct((_B, _L), jnp.float32),
            jax.ShapeDtypeStruct((_B, _B), jnp.int32),
        ),
    )(t_f.reshape(_B, 1), t_f.reshape(1, _B), latents)


def _coarse_kernel(y_ref, bins_ref, coarse_ref, sacc_ref):
    m = pl.program_id(0)

    @pl.when(m == 0)
    def _init():
        sacc_ref[...] = jnp.zeros((_B, _B), jnp.float32)

    yb = y_ref[...]
    sacc = sacc_ref[...]
    for s in range(_CB // _C):
        ys = yb[:, s * _C:(s + 1) * _C]
        ysb = ys.astype(jnp.bfloat16)
        g = lax.dot_general(ysb, ysb, (((1,), (1,)), ((), ())),
                            preferred_element_type=jnp.float32)
        n = jnp.sum(ys * ys, axis=1, keepdims=True)
        d = n + jnp.transpose(n) - 2.0 * g
        mask = bins_ref[...] < (m * _CB + s * _C)
        sacc = sacc + jnp.where(mask, d, 0.0)
    sacc_ref[...] = sacc

    @pl.when(m == _NCB - 1)
    def _fin():
        coarse_ref[0, 0] = jnp.sum(sacc_ref[...])


def _coarse_call(y, bins):
    return pl.pallas_call(
        _coarse_kernel,
        grid=(_NCB,),
        in_specs=[
            pl.BlockSpec((_B, _CB), lambda m: (0, m)),
            pl.BlockSpec((_B, _B), lambda m: (0, 0)),
        ],
        out_specs=pl.BlockSpec(memory_space=pltpu.SMEM),
        out_shape=jax.ShapeDtypeStruct((1, 1), jnp.float32),
        scratch_shapes=[pltpu.VMEM((_B, _B), jnp.float32)],
    )(y, bins)


def _sc_body(ytab_hbm, bins_hbm, out_hbm,
             bins_v, idx2_v, yrow_v, buf_b0, buf_b1,
             acc_v, acc4_v, sb0, sb1):
    cid = lax.axis_index("c")
    sid = lax.axis_index("s")
    wid = cid * 16 + sid
    base = wid * _PPW
    pltpu.sync_copy(bins_hbm.at[pl.ds(base, _PPW)], bins_v)
    pltpu.sync_copy(ytab_hbm.at[pl.ds(wid * 64, 64)], yrow_v)

    lane = lax.iota(jnp.int32, 16)
    jmask = jnp.int32(_B - 1)

    def idx_step(g, carry):
        sl = pl.ds(g * 16, 16)
        p16 = g * 16 + lane
        b16 = bins_v[sl]
        mcb16 = lax.shift_right_logical(b16, 7)
        j16 = p16 & jmask
        idx2_v[sl] = j16 * _NCB + mcb16
        return carry

    lax.fori_loop(0, _PPW // 16, idx_step, 0)

    bufs = ((buf_b0, sb0), (buf_b1, sb1))

    def fire(c):
        bb, sb = bufs[c % 2]
        csl = pl.ds(c * _CPAIRS, _CPAIRS)
        return pltpu.async_copy(ytab_hbm.at[idx2_v.at[csl]], bb, sb)

    def process(c):
        bb, _ = bufs[c % 2]

        def group_step(g, carry):
            gp = c * (_CPAIRS // 16) + g
            i_here = wid * 4 + lax.shift_right_logical(gp, 3)
            j0 = (gp & 7) * 16

            @pl.when(j0 + 15 > i_here)
            def _do_group():
                a0 = acc4_v[0, :]
                a1 = acc4_v[1, :]
                a2 = acc4_v[2, :]
                a3 = acc4_v[3, :]
                sl = pl.ds(c * _CPAIRS + g * 16, 16)
                b16 = bins_v[sl]
                irow = lax.shift_right_logical(c * _CPAIRS + g * 16, 7) * 16
                for u in range(16):
                    p = g * 16 + u
                    b = b16[u]
                    mcb = lax.shift_right_logical(b, 7)
                    q = (lax.shift_right_logical(b, 5) & 3) * _C
                    r = jnp.where(j0 + u > i_here, b & (_C - 1), 64)
                    yi0 = yrow_v[irow + mcb, pl.ds(q, 16)]
                    yj0 = bb[p, pl.ds(q, 16)]
                    yi1 = yrow_v[irow + mcb, pl.ds(q + 16, 16)]
                    yj1 = bb[p, pl.ds(q + 16, 16)]
                    d0 = yi0 - yj0
                    d1 = yi1 - yj1
                    v0 = jnp.where(lane >= r, d0 * d0, 0.0)
                    v1 = jnp.where(lane + 16 >= r, d1 * d1, 0.0)
                    if u % 2 == 0:
                        a0 = a0 + v0
                        a1 = a1 + v1
                    else:
                        a2 = a2 + v0
                        a3 = a3 + v1
                acc4_v[0, :] = a0
                acc4_v[1, :] = a1
                acc4_v[2, :] = a2
                acc4_v[3, :] = a3

            return carry

        lax.fori_loop(0, _CPAIRS // 16, group_step, 0)

    zero = jnp.zeros((16,), jnp.float32)
    for z in range(4):
        acc4_v[z, :] = zero
    pend = [fire(0), fire(1)]
    for c in range(_NCHUNK):
        pend[c % 2].wait()
        process(c)
        if c + 2 < _NCHUNK:
            pend[c % 2] = fire(c + 2)

    acc_v[...] = (acc4_v[0, :] + acc4_v[1, :]
                  + acc4_v[2, :] + acc4_v[3, :])
    pltpu.sync_copy(acc_v, out_hbm.at[wid])


def _sc_call(ytable, bins_flat):
    mesh = plsc.VectorSubcoreMesh(core_axis_name="c", subcore_axis_name="s")
    f = pl.kernel(
        _sc_body,
        mesh=mesh,
        out_type=jax.ShapeDtypeStruct((_NW, 16), jnp.float32),
        scratch_types=[
            pltpu.VMEM((_PPW,), jnp.int32),
            pltpu.VMEM((_PPW,), jnp.int32),
            pltpu.VMEM((64, _CB), jnp.float32),
            pltpu.VMEM((_CPAIRS, _CB), jnp.float32),
            pltpu.VMEM((_CPAIRS, _CB), jnp.float32),
            pltpu.VMEM((16,), jnp.float32),
            pltpu.VMEM((4, 16), jnp.float32),
            pltpu.SemaphoreType.DMA,
            pltpu.SemaphoreType.DMA,
        ],
    )
    return f(ytable, bins_flat)


def kernel(latents, time_steps):
    y, bins = _prod_call(latents, time_steps)
    ytable = y.reshape(_B * _NCB, _CB)
    sc_out = _sc_call(ytable, bins.reshape(_B * _B))
    coarse = _coarse_call(y, bins)
    total = coarse[0, 0] + 2.0 * jnp.sum(sc_out)
    return total / jnp.float32(_B * _B)


# speedup vs baseline: 1.0244x; 1.0244x over previous
import jax
import jax.numpy as jnp
from jax import lax
from jax.experimental import pallas as pl
from jax.experimental.pallas import tpu as pltpu
from jax.experimental.pallas import tpu_sc as plsc

_B = 128
_L = 2048
_C = 32
_CB = 128
_NCB = _L // _CB
_NW = 32
_PPW = _B * _B // _NW
_NCHUNK = 4
_CPAIRS = _PPW // _NCHUNK


def _prod_kernel(tcol_ref, trow_ref, lat_ref, y_ref, bins_ref):
    tom = jnp.abs(tcol_ref[...] - trow_ref[...]) + 1.0
    b = jnp.ceil(jnp.log2(tom) * 128.0)
    bins_ref[...] = jnp.clip(b, 0.0, float(_L)).astype(jnp.int32)

    k = jax.lax.broadcasted_iota(jnp.int32, (1, _L), 1).astype(jnp.float32)
    norm = jnp.exp2((k + 1.0) / 128.0) - jnp.exp2(k / 128.0)
    norm_last = jnp.exp2(jnp.float32(_L) / 128.0) - jnp.exp2(
        (jnp.float32(_L) - 1.0) / 128.0)
    y_ref[...] = lat_ref[...] * jnp.sqrt(norm_last / norm)


def _prod_call(latents, time_steps):
    t_f = time_steps.astype(jnp.float32)
    return pl.pallas_call(
        _prod_kernel,
        out_shape=(
            jax.ShapeDtypeStruct((_B, _L), jnp.float32),
            jax.ShapeDtypeStruct((_B, _B), jnp.int32),
        ),
    )(t_f.reshape(_B, 1), t_f.reshape(1, _B), latents)


def _coarse_kernel(y_ref, bins_ref, coarse_ref, sacc_ref):
    m = pl.program_id(0)

    @pl.when(m == 0)
    def _init():
        sacc_ref[...] = jnp.zeros((_B, _B), jnp.float32)

    yb = y_ref[...]
    ones_row = jnp.ones((1, _C), jnp.float32)
    parts = []
    for s in range(_CB // _C):
        ys = yb[:, s * _C:(s + 1) * _C]
        ysb = ys.astype(jnp.bfloat16)
        g = lax.dot_general(ysb, ysb, (((1,), (1,)), ((), ())),
                            preferred_element_type=jnp.float32)
        ys2 = ys * ys
        n_col = lax.dot_general(ys2, ones_row, (((1,), (1,)), ((), ())),
                                preferred_element_type=jnp.float32)
        n_row = lax.dot_general(ones_row, ys2, (((1,), (1,)), ((), ())),
                                preferred_element_type=jnp.float32)
        d = n_col + n_row - 2.0 * g
        mask = bins_ref[...] < (m * _CB + s * _C)
        parts.append(jnp.where(mask, d, 0.0))
    sacc_ref[...] = (sacc_ref[...]
                     + (parts[0] + parts[1]) + (parts[2] + parts[3]))

    @pl.when(m == _NCB - 1)
    def _fin():
        coarse_ref[0, 0] = jnp.sum(sacc_ref[...])


def _coarse_call(y, bins):
    return pl.pallas_call(
        _coarse_kernel,
        grid=(_NCB,),
        in_specs=[
            pl.BlockSpec((_B, _CB), lambda m: (0, m)),
            pl.BlockSpec((_B, _B), lambda m: (0, 0)),
        ],
        out_specs=pl.BlockSpec(memory_space=pltpu.SMEM),
        out_shape=jax.ShapeDtypeStruct((1, 1), jnp.float32),
        scratch_shapes=[pltpu.VMEM((_B, _B), jnp.float32)],
    )(y, bins)


def _sc_body(ytab_hbm, bins_hbm, out_hbm,
             bins_v, idx2_v, yrow_v, buf_b0, buf_b1,
             acc_v, acc4_v, sb0, sb1):
    cid = lax.axis_index("c")
    sid = lax.axis_index("s")
    wid = cid * 16 + sid
    base = wid * _PPW
    pltpu.sync_copy(bins_hbm.at[pl.ds(base, _PPW)], bins_v)
    pltpu.sync_copy(ytab_hbm.at[pl.ds(wid * 64, 64)], yrow_v)

    lane = lax.iota(jnp.int32, 16)
    jmask = jnp.int32(_B - 1)

    def idx_step(g, carry):
        sl = pl.ds(g * 16, 16)
        p16 = g * 16 + lane
        b16 = bins_v[sl]
        mcb16 = lax.shift_right_logical(b16, 7)
        j16 = p16 & jmask
        idx2_v[sl] = j16 * _NCB + mcb16
        return carry

    lax.fori_loop(0, _PPW // 16, idx_step, 0)

    bufs = ((buf_b0, sb0), (buf_b1, sb1))

    def fire(c):
        bb, sb = bufs[c % 2]
        csl = pl.ds(c * _CPAIRS, _CPAIRS)
        return pltpu.async_copy(ytab_hbm.at[idx2_v.at[csl]], bb, sb)

    def process(c):
        bb, _ = bufs[c % 2]

        def group_step(g, carry):
            gp = c * (_CPAIRS // 16) + g
            i_here = wid * 4 + lax.shift_right_logical(gp, 3)
            j0 = (gp & 7) * 16

            @pl.when(j0 + 15 > i_here)
            def _do_group():
                a0 = acc4_v[0, :]
                a1 = acc4_v[1, :]
                a2 = acc4_v[2, :]
                a3 = acc4_v[3, :]
                sl = pl.ds(c * _CPAIRS + g * 16, 16)
                b16 = bins_v[sl]
                irow = lax.shift_right_logical(c * _CPAIRS + g * 16, 7) * 16
                for u in range(16):
                    p = g * 16 + u
                    b = b16[u]
                    mcb = lax.shift_right_logical(b, 7)
                    q = (lax.shift_right_logical(b, 5) & 3) * _C
                    r = jnp.where(j0 + u > i_here, b & (_C - 1), 64)
                    yi0 = yrow_v[irow + mcb, pl.ds(q, 16)]
                    yj0 = bb[p, pl.ds(q, 16)]
                    yi1 = yrow_v[irow + mcb, pl.ds(q + 16, 16)]
                    yj1 = bb[p, pl.ds(q + 16, 16)]
                    d0 = yi0 - yj0
                    d1 = yi1 - yj1
                    v0 = jnp.where(lane >= r, d0 * d0, 0.0)
                    v1 = jnp.where(lane + 16 >= r, d1 * d1, 0.0)
                    if u % 2 == 0:
                        a0 = a0 + v0
                        a1 = a1 + v1
                    else:
                        a2 = a2 + v0
                        a3 = a3 + v1
                acc4_v[0, :] = a0
                acc4_v[1, :] = a1
                acc4_v[2, :] = a2
                acc4_v[3, :] = a3

            return carry

        lax.fori_loop(0, _CPAIRS // 16, group_step, 0)

    zero = jnp.zeros((16,), jnp.float32)
    for z in range(4):
        acc4_v[z, :] = zero
    pend = [fire(0), fire(1)]
    for c in range(_NCHUNK):
        pend[c % 2].wait()
        process(c)
        if c + 2 < _NCHUNK:
            pend[c % 2] = fire(c + 2)

    acc_v[...] = (acc4_v[0, :] + acc4_v[1, :]
                  + acc4_v[2, :] + acc4_v[3, :])
    pltpu.sync_copy(acc_v, out_hbm.at[wid])


def _sc_call(ytable, bins_flat):
    mesh = plsc.VectorSubcoreMesh(core_axis_name="c", subcore_axis_name="s")
    f = pl.kernel(
        _sc_body,
        mesh=mesh,
        out_type=jax.ShapeDtypeStruct((_NW, 16), jnp.float32),
        scratch_types=[
            pltpu.VMEM((_PPW,), jnp.int32),
            pltpu.VMEM((_PPW,), jnp.int32),
            pltpu.VMEM((64, _CB), jnp.float32),
            pltpu.VMEM((_CPAIRS, _CB), jnp.float32),
            pltpu.VMEM((_CPAIRS, _CB), jnp.float32),
            pltpu.VMEM((16,), jnp.float32),
            pltpu.VMEM((4, 16), jnp.float32),
            pltpu.SemaphoreType.DMA,
            pltpu.SemaphoreType.DMA,
        ],
    )
    return f(ytable, bins_flat)


def kernel(latents, time_steps):
    y, bins = _prod_call(latents, time_steps)
    ytable = y.reshape(_B * _NCB, _CB)
    sc_out = _sc_call(ytable, bins.reshape(_B * _B))
    coarse = _coarse_call(y, bins)
    total = coarse[0, 0] + 2.0 * jnp.sum(sc_out)
    return total / jnp.float32(_B * _B)


# speedup vs baseline: 1.0313x; 1.0068x over previous
import jax
import jax.numpy as jnp
from jax import lax
from jax.experimental import pallas as pl
from jax.experimental.pallas import tpu as pltpu
from jax.experimental.pallas import tpu_sc as plsc

_B = 128
_L = 2048
_C = 32
_CB = 128
_NCB = _L // _CB
_NW = 32
_PPW = _B * _B // _NW
_NCHUNK = 4
_CPAIRS = _PPW // _NCHUNK
_TCB = 256
_NTC = _L // _TCB


def _prod_kernel(tcol_ref, trow_ref, lat_ref, y_ref, bins_ref):
    tom = jnp.abs(tcol_ref[...] - trow_ref[...]) + 1.0
    b = jnp.ceil(jnp.log2(tom) * 128.0)
    bins_ref[...] = jnp.clip(b, 0.0, float(_L)).astype(jnp.int32)

    k = jax.lax.broadcasted_iota(jnp.int32, (1, _L), 1).astype(jnp.float32)
    norm = jnp.exp2((k + 1.0) / 128.0) - jnp.exp2(k / 128.0)
    norm_last = jnp.exp2(jnp.float32(_L) / 128.0) - jnp.exp2(
        (jnp.float32(_L) - 1.0) / 128.0)
    y_ref[...] = lat_ref[...] * jnp.sqrt(norm_last / norm)


def _prod_call(latents, time_steps):
    t_f = time_steps.astype(jnp.float32)
    return pl.pallas_call(
        _prod_kernel,
        out_shape=(
            jax.ShapeDtypeStruct((_B, _L), jnp.float32),
            jax.ShapeDtypeStruct((_B, _B), jnp.int32),
        ),
    )(t_f.reshape(_B, 1), t_f.reshape(1, _B), latents)


def _coarse_kernel(y_ref, bins_ref, coarse_ref, sacc_ref):
    m = pl.program_id(0)

    @pl.when(m == 0)
    def _init():
        sacc_ref[...] = jnp.zeros((_B, _B), jnp.float32)

    yb = y_ref[...]
    ones_row = jnp.ones((1, _C), jnp.float32)
    parts = []
    for s in range(_TCB // _C):
        ys = yb[:, s * _C:(s + 1) * _C]
        ysb = ys.astype(jnp.bfloat16)
        g = lax.dot_general(ysb, ysb, (((1,), (1,)), ((), ())),
                            preferred_element_type=jnp.float32)
        ys2 = ys * ys
        n_col = lax.dot_general(ys2, ones_row, (((1,), (1,)), ((), ())),
                                preferred_element_type=jnp.float32)
        n_row = lax.dot_general(ones_row, ys2, (((1,), (1,)), ((), ())),
                                preferred_element_type=jnp.float32)
        d = n_col + n_row - 2.0 * g
        mask = bins_ref[...] < (m * _TCB + s * _C)
        parts.append(jnp.where(mask, d, 0.0))
    psum = parts[0]
    for pp in parts[1:]:
        psum = psum + pp
    sacc_ref[...] = sacc_ref[...] + psum

    @pl.when(m == _NTC - 1)
    def _fin():
        coarse_ref[0, 0] = jnp.sum(sacc_ref[...])


def _coarse_call(y, bins):
    return pl.pallas_call(
        _coarse_kernel,
        grid=(_NTC,),
        in_specs=[
            pl.BlockSpec((_B, _TCB), lambda m: (0, m)),
            pl.BlockSpec((_B, _B), lambda m: (0, 0)),
        ],
        out_specs=pl.BlockSpec(memory_space=pltpu.SMEM),
        out_shape=jax.ShapeDtypeStruct((1, 1), jnp.float32),
        scratch_shapes=[pltpu.VMEM((_B, _B), jnp.float32)],
    )(y, bins)


def _sc_body(ytab_hbm, bins_hbm, out_hbm,
             bins_v, idx2_v, yrow_v, buf_b0, buf_b1,
             acc_v, acc4_v, sb0, sb1):
    cid = lax.axis_index("c")
    sid = lax.axis_index("s")
    wid = cid * 16 + sid
    base = wid * _PPW
    pltpu.sync_copy(bins_hbm.at[pl.ds(base, _PPW)], bins_v)
    pltpu.sync_copy(ytab_hbm.at[pl.ds(wid * 64, 64)], yrow_v)

    lane = lax.iota(jnp.int32, 16)
    jmask = jnp.int32(_B - 1)

    def idx_step(g, carry):
        sl = pl.ds(g * 16, 16)
        p16 = g * 16 + lane
        b16 = bins_v[sl]
        mcb16 = lax.shift_right_logical(b16, 7)
        j16 = p16 & jmask
        idx2_v[sl] = j16 * _NCB + mcb16
        return carry

    lax.fori_loop(0, _PPW // 16, idx_step, 0)

    bufs = ((buf_b0, sb0), (buf_b1, sb1))

    def fire(c):
        bb, sb = bufs[c % 2]
        csl = pl.ds(c * _CPAIRS, _CPAIRS)
        return pltpu.async_copy(ytab_hbm.at[idx2_v.at[csl]], bb, sb)

    def process(c):
        bb, _ = bufs[c % 2]

        def group_step(g, carry):
            gp = c * (_CPAIRS // 16) + g
            i_here = wid * 4 + lax.shift_right_logical(gp, 3)
            j0 = (gp & 7) * 16

            @pl.when(j0 + 15 > i_here)
            def _do_group():
                a0 = acc4_v[0, :]
                a1 = acc4_v[1, :]
                a2 = acc4_v[2, :]
                a3 = acc4_v[3, :]
                sl = pl.ds(c * _CPAIRS + g * 16, 16)
                b16 = bins_v[sl]
                irow = lax.shift_right_logical(c * _CPAIRS + g * 16, 7) * 16
                for u in range(16):
                    p = g * 16 + u
                    b = b16[u]
                    mcb = lax.shift_right_logical(b, 7)
                    q = (lax.shift_right_logical(b, 5) & 3) * _C
                    r = jnp.where(j0 + u > i_here, b & (_C - 1), 64)
                    yi0 = yrow_v[irow + mcb, pl.ds(q, 16)]
                    yj0 = bb[p, pl.ds(q, 16)]
                    yi1 = yrow_v[irow + mcb, pl.ds(q + 16, 16)]
                    yj1 = bb[p, pl.ds(q + 16, 16)]
                    d0 = yi0 - yj0
                    d1 = yi1 - yj1
                    v0 = jnp.where(lane >= r, d0 * d0, 0.0)
                    v1 = jnp.where(lane + 16 >= r, d1 * d1, 0.0)
                    if u % 2 == 0:
                        a0 = a0 + v0
                        a1 = a1 + v1
                    else:
                        a2 = a2 + v0
                        a3 = a3 + v1
                acc4_v[0, :] = a0
                acc4_v[1, :] = a1
                acc4_v[2, :] = a2
                acc4_v[3, :] = a3

            return carry

        lax.fori_loop(0, _CPAIRS // 16, group_step, 0)

    zero = jnp.zeros((16,), jnp.float32)
    for z in range(4):
        acc4_v[z, :] = zero
    pend = [fire(0), fire(1)]
    for c in range(_NCHUNK):
        pend[c % 2].wait()
        process(c)
        if c + 2 < _NCHUNK:
            pend[c % 2] = fire(c + 2)

    acc_v[...] = (acc4_v[0, :] + acc4_v[1, :]
                  + acc4_v[2, :] + acc4_v[3, :])
    pltpu.sync_copy(acc_v, out_hbm.at[wid])


def _sc_call(ytable, bins_flat):
    mesh = plsc.VectorSubcoreMesh(core_axis_name="c", subcore_axis_name="s")
    f = pl.kernel(
        _sc_body,
        mesh=mesh,
        out_type=jax.ShapeDtypeStruct((_NW, 16), jnp.float32),
        scratch_types=[
            pltpu.VMEM((_PPW,), jnp.int32),
            pltpu.VMEM((_PPW,), jnp.int32),
            pltpu.VMEM((64, _CB), jnp.float32),
            pltpu.VMEM((_CPAIRS, _CB), jnp.float32),
            pltpu.VMEM((_CPAIRS, _CB), jnp.float32),
            pltpu.VMEM((16,), jnp.float32),
            pltpu.VMEM((4, 16), jnp.float32),
            pltpu.SemaphoreType.DMA,
            pltpu.SemaphoreType.DMA,
        ],
    )
    return f(ytable, bins_flat)


def kernel(latents, time_steps):
    y, bins = _prod_call(latents, time_steps)
    ytable = y.reshape(_B * _NCB, _CB)
    sc_out = _sc_call(ytable, bins.reshape(_B * _B))
    coarse = _coarse_call(y, bins)
    total = coarse[0, 0] + 2.0 * jnp.sum(sc_out)
    return total / jnp.float32(_B * _B)
